# baseline (speedup 1.0000x reference)
import jax
import jax.numpy as jnp
from jax import lax
from jax.experimental import pallas as pl
from jax.experimental.pallas import tpu as pltpu
from jax.experimental.pallas import tpu_sc as plsc

B, T, D = 16, 4096, 1024
HALF = B // 2


def _last_row_gather(flat_hbm, seqlens_hbm, out_hbm, idx_v, rows_v, sem):
    s = lax.axis_index("s")

    @pl.when(s < 2)
    def _():
        base = s * HALF
        pltpu.sync_copy(seqlens_hbm, idx_v)
        idx_v[...] = idx_v[...] - 1 + lax.iota(jnp.int32, B) * T
        gather = pltpu.async_copy(
            flat_hbm.at[idx_v.at[pl.ds(base, HALF)]], rows_v, sem
        )
        gather.wait()
        pltpu.sync_copy(rows_v, out_hbm.at[pl.ds(base, HALF)])


def kernel(inputs, seqlens):
    flat = inputs.reshape(B * T, D)
    mesh = plsc.VectorSubcoreMesh(
        core_axis_name="c", subcore_axis_name="s", num_cores=1, num_subcores=2
    )
    k = pl.kernel(
        _last_row_gather,
        mesh=mesh,
        out_type=jax.ShapeDtypeStruct((B, D), jnp.float32),
        scratch_types=[
            pltpu.VMEM((B,), jnp.int32),
            pltpu.VMEM((HALF, D), jnp.float32),
            pltpu.SemaphoreType.DMA,
        ],
    )
    return k(flat, seqlens)


# speedup vs baseline: 1.0149x; 1.0149x over previous
import jax
import jax.numpy as jnp
from jax import lax
from jax.experimental import pallas as pl
from jax.experimental.pallas import tpu as pltpu
from jax.experimental.pallas import tpu_sc as plsc

B, T, D = 16, 4096, 1024


def _body(flat_hbm, seqlens_hbm, out_hbm, sl_smem, sem):
    c = lax.axis_index("c")

    @pl.when(c == 0)
    def _():
        pltpu.sync_copy(seqlens_hbm, sl_smem)
        copies = []
        for b in range(B):
            s = sl_smem[b]
            cp = pltpu.make_async_copy(
                flat_hbm.at[pl.ds(b * T + s - 1, 1)],
                out_hbm.at[pl.ds(b, 1)],
                sem,
            )
            cp.start()
            copies.append(cp)
        for cp in copies:
            cp.wait()


def kernel(inputs, seqlens):
    flat = inputs.reshape(B * T, D)
    mesh = plsc.ScalarSubcoreMesh(axis_name="c", num_cores=1)
    k = pl.kernel(
        _body,
        mesh=mesh,
        out_type=jax.ShapeDtypeStruct((B, D), jnp.float32),
        scratch_types=[
            pltpu.SMEM((B,), jnp.int32),
            pltpu.SemaphoreType.DMA,
        ],
    )
    return k(flat, seqlens)
